# baseline (speedup 1.0000x reference)
import functools

import jax
import jax.numpy as jnp
from jax import lax
from jax.experimental import pallas as pl
from jax.experimental.pallas import tpu as pltpu
from jax.experimental.pallas import tpu_sc as plsc

NC = 2
NS = 16
NW = NC * NS
L = 16
CH = 128
BC = 4
H = 32


def _encode_body(x_ref, w1t_ref, b1_ref, t_ref, nrm_ref):
    h = jnp.dot(x_ref[...], w1t_ref[...], preferred_element_type=jnp.float32)
    h = jnp.maximum(h + b1_ref[...], 0.0)
    ss = jnp.sum(h * h, axis=1, keepdims=True)
    t_ref[...] = h
    nrm_ref[...] = jnp.maximum(jnp.sqrt(ss), 1e-12)


def _finish_body(t_ref, p0_ref, p1_ref, d0_ref, d1_ref, par_ref, w2t_ref,
                 b2_ref, o_ref):
    h = t_ref[...]
    b = par_ref[0, 0]
    s = par_ref[0, 1]
    ss = jnp.sum(h * h, axis=1, keepdims=True)
    cs = ss / jnp.maximum(ss, 1e-24)
    wself = jnp.exp(b * cs - s)
    num = p0_ref[...] + p1_ref[...] + wself * h
    den = d0_ref[...] + d1_ref[...] + wself
    res = num / den
    z = jnp.dot(res, w2t_ref[...], preferred_element_type=jnp.float32) + b2_ref[...]
    m = jnp.max(z, axis=1, keepdims=True)
    lse = m + jnp.log(jnp.sum(jnp.exp(z - m), axis=1, keepdims=True))
    o_ref[...] = z - lse


def _make_edge_kernel(n2, nchunk):
    rpt = n2 // NS
    nblk = nchunk // BC
    mesh = plsc.VectorSubcoreMesh(core_axis_name="c", subcore_axis_name="s")

    @functools.partial(
        pl.kernel,
        out_type=[
            jax.ShapeDtypeStruct((NC, n2, H), jnp.float32),
            jax.ShapeDtypeStruct((NC, n2), jnp.float32),
        ],
        mesh=mesh,
        compiler_params=pltpu.CompilerParams(
            needs_layout_passes=False, use_tc_tiling_on_sc=False),
        scratch_types=[
            pltpu.VMEM_SHARED((n2, H), jnp.float32),
            pltpu.VMEM_SHARED((n2,), jnp.float32),
            pltpu.VMEM((2 * BC, CH), jnp.int32),
            pltpu.VMEM((2 * BC, CH), jnp.int32),
            pltpu.VMEM((CH, H), jnp.float32),
            pltpu.VMEM((CH, H), jnp.float32),
            pltpu.VMEM((CH, H), jnp.float32),
            pltpu.VMEM((CH, H), jnp.float32),
            pltpu.VMEM((CH,), jnp.float32),
            pltpu.VMEM((CH,), jnp.float32),
            pltpu.VMEM((CH,), jnp.float32),
            pltpu.VMEM((CH,), jnp.float32),
            pltpu.VMEM((CH,), jnp.float32),
            pltpu.VMEM((CH,), jnp.float32),
            pltpu.VMEM((2 * L,), jnp.float32),
            pltpu.SemaphoreType.DMA,
            pltpu.SemaphoreType.DMA,
        ],
    )
    def edge_kernel(t_hbm, nrm_hbm, idx_hbm, par_hbm, outn_hbm, outd_hbm,
                    accn, accd, idx0, idx1, src0, src1, dst0, dst1,
                    ns0, ns1, nd0, nd1, w0, w1, par, sem0, sem1):
        c = lax.axis_index("c")
        s = lax.axis_index("s")
        wid = c * NS + s

        idxb = (idx0, idx1)
        srcb = (src0, src1)
        dstb = (dst0, dst1)
        nsb = (ns0, ns1)
        ndb = (nd0, nd1)
        wb = (w0, w1)
        semb = (sem0, sem1)

        zv = jnp.zeros((L,), jnp.float32)

        @pl.loop(0, CH)
        def _(r):
            dst0[r, pl.ds(0, L)] = zv
            dst0[r, pl.ds(L, L)] = zv

        for g in range(CH // L):
            w0[pl.ds(g * L, L)] = zv

        row0 = s * rpt
        nfull = rpt // CH
        rem = rpt - nfull * CH

        @pl.loop(0, nfull)
        def _(i):
            pltpu.sync_copy(dst0, accn.at[pl.ds(row0 + i * CH, CH)])
            pltpu.sync_copy(w0, accd.at[pl.ds(row0 + i * CH, CH)])

        if rem:
            pltpu.sync_copy(dst0.at[pl.ds(0, rem)],
                            accn.at[pl.ds(row0 + nfull * CH, rem)])
            pltpu.sync_copy(w0.at[pl.ds(0, rem)],
                            accd.at[pl.ds(row0 + nfull * CH, rem)])

        pltpu.sync_copy(par_hbm, par)
        plsc.subcore_barrier()

        bvec = par[pl.ds(0, L)]
        svec = par[pl.ds(L, L)]
        riota = lax.iota(jnp.int32, L)
        ibase = wid * (2 * nchunk)

        def fire(jrow, lrow, b):
            pltpu.async_copy(t_hbm.at[idxb[jrow].at[2 * lrow]],
                             srcb[b], semb[b])
            pltpu.async_copy(t_hbm.at[idxb[jrow].at[2 * lrow + 1]],
                             dstb[b], semb[b])
            pltpu.async_copy(nrm_hbm.at[idxb[jrow].at[2 * lrow]],
                             nsb[b], semb[b])
            pltpu.async_copy(nrm_hbm.at[idxb[jrow].at[2 * lrow + 1]],
                             ndb[b], semb[b])

        def process(jrow, lrow, b):
            pltpu.make_async_copy(
                t_hbm.at[pl.ds(0, CH)], srcb[b], semb[b]).wait()
            pltpu.make_async_copy(
                t_hbm.at[pl.ds(0, CH)], dstb[b], semb[b]).wait()
            pltpu.make_async_copy(
                nrm_hbm.at[pl.ds(0, CH)], nsb[b], semb[b]).wait()
            pltpu.make_async_copy(
                nrm_hbm.at[pl.ds(0, CH)], ndb[b], semb[b]).wait()

            @pl.loop(0, CH // L)
            def _(g):
                rowi = riota + (g * L)
                dot = None
                for k in range(H):
                    ck = jnp.full((L,), k, jnp.int32)
                    p = (plsc.load_gather(srcb[b], [rowi, ck]) *
                         plsc.load_gather(dstb[b], [rowi, ck]))
                    dot = p if dot is None else dot + p
                nn = nsb[b][pl.ds(g * L, L)] * ndb[b][pl.ds(g * L, L)]
                w = jnp.exp(bvec * (dot / nn) - svec)
                for k in range(H):
                    ck = jnp.full((L,), k, jnp.int32)
                    v = w * plsc.load_gather(srcb[b], [rowi, ck])
                    plsc.store_scatter(dstb[b], [rowi, ck], v)
                wb[b][pl.ds(g * L, L)] = w

            pltpu.sync_copy(dstb[b], accn.at[idxb[jrow].at[2 * lrow + 1]],
                            add=True)
            pltpu.sync_copy(wb[b], accd.at[idxb[jrow].at[2 * lrow + 1]],
                            add=True)

        pltpu.sync_copy(idx_hbm.at[pl.ds(ibase, 2 * BC)], idx0)
        fire(0, 0, 0)
        fire(0, 1, 1)

        @pl.loop(0, nblk // 2)
        def _(i):
            for mm in range(2):
                m = 2 * i + mm
                jb = m * BC

                @pl.when(m + 1 < nblk)
                def _():
                    pltpu.sync_copy(
                        idx_hbm.at[pl.ds(ibase + (m + 1) * 2 * BC, 2 * BC)],
                        idxb[1 - mm])

                for cc in range(BC):
                    b = cc % 2
                    process(mm, cc, b)
                    nxt = cc + 2

                    @pl.when(jb + nxt < nchunk)
                    def _():
                        if nxt < BC:
                            fire(mm, nxt, b)
                        else:
                            fire(1 - mm, nxt - BC, b)

        plsc.subcore_barrier()
        pltpu.sync_copy(accn.at[pl.ds(row0, rpt)],
                        outn_hbm.at[c, pl.ds(row0, rpt)])
        pltpu.sync_copy(accd.at[pl.ds(row0, rpt)],
                        outd_hbm.at[c, pl.ds(row0, rpt)])

    return edge_kernel


def kernel(x, edge_index, W1, b1, beta, W2, b2):
    n, d = x.shape
    e = edge_index.shape[1]
    n2 = ((n + NS * 8 - 1) // (NS * 8)) * (NS * 8)
    step = NW * CH * 2 * BC
    e_pad = ((e + step - 1) // step) * step
    ept = e_pad // NW
    nchunk = ept // CH

    nb = 25
    bn = n // nb
    t, nrm = pl.pallas_call(
        _encode_body,
        grid=(nb,),
        in_specs=[
            pl.BlockSpec((bn, d), lambda i: (i, 0)),
            pl.BlockSpec((d, H), lambda i: (0, 0)),
            pl.BlockSpec((1, H), lambda i: (0, 0)),
        ],
        out_specs=[
            pl.BlockSpec((bn, H), lambda i: (i, 0)),
            pl.BlockSpec((bn, 1), lambda i: (i, 0)),
        ],
        out_shape=[
            jax.ShapeDtypeStruct((n, H), jnp.float32),
            jax.ShapeDtypeStruct((n, 1), jnp.float32),
        ],
    )(x, W1.T, b1.reshape(1, H))
    t_full = jnp.pad(t, ((0, n2 - n), (0, 0)))
    nrm_full = jnp.pad(nrm.reshape(n), (0, n2 - n), constant_values=1.0)

    src = jnp.concatenate(
        [edge_index[0], jnp.zeros((e_pad - e,), jnp.int32)])
    dst = jnp.concatenate(
        [edge_index[1], jnp.full((e_pad - e,), n2 - 1, jnp.int32)])
    idx = jnp.stack([src.reshape(NW, nchunk, CH),
                     dst.reshape(NW, nchunk, CH)], axis=2)
    idx = idx.reshape(NW * nchunk * 2, CH)

    bf = beta.astype(jnp.float32)
    par = jnp.concatenate(
        [jnp.full((L,), bf), jnp.full((L,), jnp.abs(bf))])

    pn, pd = _make_edge_kernel(n2, nchunk)(t_full, nrm_full, idx, par)

    par2 = jnp.stack([bf, jnp.abs(bf)]).reshape(1, 2)
    out = pl.pallas_call(
        _finish_body,
        grid=(nb,),
        in_specs=[
            pl.BlockSpec((bn, H), lambda i: (i, 0)),
            pl.BlockSpec((bn, H), lambda i: (i, 0)),
            pl.BlockSpec((bn, H), lambda i: (i, 0)),
            pl.BlockSpec((bn, 1), lambda i: (i, 0)),
            pl.BlockSpec((bn, 1), lambda i: (i, 0)),
            pl.BlockSpec((1, 2), lambda i: (0, 0)),
            pl.BlockSpec((H, 2), lambda i: (0, 0)),
            pl.BlockSpec((1, 2), lambda i: (0, 0)),
        ],
        out_specs=pl.BlockSpec((bn, 2), lambda i: (i, 0)),
        out_shape=jax.ShapeDtypeStruct((n, 2), jnp.float32),
    )(t, pn[0, :n], pn[1, :n], pd[0, :n].reshape(n, 1),
      pd[1, :n].reshape(n, 1), par2, W2.T, b2.reshape(1, 2))
    return out


# speedup vs baseline: 1.4234x; 1.4234x over previous
import functools

import jax
import jax.numpy as jnp
from jax import lax
from jax.experimental import pallas as pl
from jax.experimental.pallas import tpu as pltpu
from jax.experimental.pallas import tpu_sc as plsc

NC = 2
NS = 16
NW = NC * NS
L = 16
CH = 112
BC = 4
H = 32
RS = 40


def _encode_body(x_ref, w1t_ref, b1_ref, t40_ref, t32_ref):
    h = jnp.dot(x_ref[...], w1t_ref[...], preferred_element_type=jnp.float32)
    h = jnp.maximum(h + b1_ref[...], 0.0)
    ss = jnp.sum(h * h, axis=1, keepdims=True)
    norm = jnp.sqrt(ss)
    hn = h / jnp.maximum(norm, 1e-12)
    pad = jnp.zeros((h.shape[0], RS - H - 1), jnp.float32)
    t40_ref[...] = jnp.concatenate([hn, norm, pad], axis=1)
    t32_ref[...] = hn


def _finish_body(t_ref, p0_ref, p1_ref, d0_ref, d1_ref, par_ref, w2t_ref,
                 b2_ref, o_ref):
    t = t_ref[...]
    hn = t[:, :H]
    norm = t[:, H:H + 1]
    b = par_ref[0, 0]
    s = par_ref[0, 1]
    cs = jnp.sum(hn * hn, axis=1, keepdims=True)
    wself = jnp.exp(b * cs - s)
    num = p0_ref[...] + p1_ref[...] + wself * (hn * norm)
    den = d0_ref[...] + d1_ref[...] + wself
    res = num / den
    z = jnp.dot(res, w2t_ref[...], preferred_element_type=jnp.float32) + b2_ref[...]
    m = jnp.max(z, axis=1, keepdims=True)
    lse = m + jnp.log(jnp.sum(jnp.exp(z - m), axis=1, keepdims=True))
    o_ref[...] = z - lse


def _make_edge_kernel(n2, nchunk):
    rpt = n2 // NS
    nblk = nchunk // BC
    mesh = plsc.VectorSubcoreMesh(core_axis_name="c", subcore_axis_name="s")

    @functools.partial(
        pl.kernel,
        out_type=[
            jax.ShapeDtypeStruct((NC, n2, H), jnp.float32),
            jax.ShapeDtypeStruct((NC, n2), jnp.float32),
        ],
        mesh=mesh,
        compiler_params=pltpu.CompilerParams(
            needs_layout_passes=False, use_tc_tiling_on_sc=False),
        scratch_types=[
            pltpu.VMEM_SHARED((n2, H), jnp.float32),
            pltpu.VMEM_SHARED((n2,), jnp.float32),
            pltpu.VMEM((2 * BC, CH), jnp.int32),
            pltpu.VMEM((2 * BC, CH), jnp.int32),
            pltpu.VMEM((CH, RS), jnp.float32),
            pltpu.VMEM((CH, RS), jnp.float32),
            pltpu.VMEM((CH, H), jnp.float32),
            pltpu.VMEM((CH, H), jnp.float32),
            pltpu.VMEM((CH, H), jnp.float32),
            pltpu.VMEM((CH, H), jnp.float32),
            pltpu.VMEM((CH,), jnp.float32),
            pltpu.VMEM((CH,), jnp.float32),
            pltpu.VMEM((2 * L,), jnp.float32),
            pltpu.SemaphoreType.DMA,
            pltpu.SemaphoreType.DMA,
            pltpu.SemaphoreType.DMA,
            pltpu.SemaphoreType.DMA,
        ],
    )
    def edge_kernel(t40_hbm, t32_hbm, idx_hbm, par_hbm, outn_hbm, outd_hbm,
                    accn, accd, idx0, idx1, src0, src1, dst0, dst1,
                    stg0, stg1, w0, w1, par, sem0, sem1, ssem0, ssem1):
        c = lax.axis_index("c")
        s = lax.axis_index("s")
        wid = c * NS + s

        idxb = (idx0, idx1)
        srcb = (src0, src1)
        dstb = (dst0, dst1)
        stgb = (stg0, stg1)
        wb = (w0, w1)
        semb = (sem0, sem1)
        ssemb = (ssem0, ssem1)

        zv = jnp.zeros((L,), jnp.float32)

        @pl.loop(0, CH)
        def _(r):
            dst0[r, pl.ds(0, L)] = zv
            dst0[r, pl.ds(L, L)] = zv

        for g in range(CH // L):
            w0[pl.ds(g * L, L)] = zv

        row0 = s * rpt
        nfull = rpt // CH
        rem = rpt - nfull * CH

        @pl.loop(0, nfull)
        def _(i):
            pltpu.sync_copy(dst0, accn.at[pl.ds(row0 + i * CH, CH)])
            pltpu.sync_copy(w0, accd.at[pl.ds(row0 + i * CH, CH)])

        if rem:
            pltpu.sync_copy(dst0.at[pl.ds(0, rem)],
                            accn.at[pl.ds(row0 + nfull * CH, rem)])
            pltpu.sync_copy(w0.at[pl.ds(0, rem)],
                            accd.at[pl.ds(row0 + nfull * CH, rem)])

        pltpu.sync_copy(par_hbm, par)
        plsc.subcore_barrier()

        bvec = par[pl.ds(0, L)]
        svec = par[pl.ds(L, L)]
        riota = lax.iota(jnp.int32, L)
        ibase = wid * (2 * nchunk)

        def fire(jrow, lrow, b):
            pltpu.async_copy(t40_hbm.at[idxb[jrow].at[2 * lrow]],
                             srcb[b], semb[b])
            pltpu.async_copy(t32_hbm.at[idxb[jrow].at[2 * lrow + 1]],
                             dstb[b], semb[b])

        def drain_scatters(b):
            pltpu.make_async_copy(
                t32_hbm.at[pl.ds(0, CH)], stgb[b], ssemb[b]).wait()
            pltpu.make_async_copy(
                outd_hbm.at[c, pl.ds(0, CH)], wb[b], ssemb[b]).wait()

        def process(jrow, lrow, b, drain):
            pltpu.make_async_copy(
                t40_hbm.at[pl.ds(0, CH)], srcb[b], semb[b]).wait()
            pltpu.make_async_copy(
                t32_hbm.at[pl.ds(0, CH)], dstb[b], semb[b]).wait()
            if drain:
                drain_scatters(b)

            @pl.loop(0, CH // L)
            def _(g):
                rowi = riota + (g * L)
                dot = None
                for k in range(H):
                    ck = jnp.full((L,), k, jnp.int32)
                    p = (plsc.load_gather(srcb[b], [rowi, ck]) *
                         plsc.load_gather(dstb[b], [rowi, ck]))
                    dot = p if dot is None else dot + p
                w = jnp.exp(bvec * dot - svec)
                cn = jnp.full((L,), H, jnp.int32)
                u = w * plsc.load_gather(srcb[b], [rowi, cn])
                for k in range(H):
                    ck = jnp.full((L,), k, jnp.int32)
                    v = u * plsc.load_gather(srcb[b], [rowi, ck])
                    plsc.store_scatter(stgb[b], [rowi, ck], v)
                wb[b][pl.ds(g * L, L)] = w

            pltpu.async_copy(stgb[b], accn.at[idxb[jrow].at[2 * lrow + 1]],
                             ssemb[b], add=True)
            pltpu.async_copy(wb[b], accd.at[idxb[jrow].at[2 * lrow + 1]],
                             ssemb[b], add=True)

        def block(m, mm, static, first=False):
            if not first:
                drain_scatters(0)
                drain_scatters(1)
            if static:
                pltpu.sync_copy(
                    idx_hbm.at[pl.ds(ibase + (m + 1) * 2 * BC, 2 * BC)],
                    idxb[1 - mm])
            else:
                @pl.when(m + 1 < nblk)
                def _():
                    pltpu.sync_copy(
                        idx_hbm.at[pl.ds(ibase + (m + 1) * 2 * BC, 2 * BC)],
                        idxb[1 - mm])

            for cc in range(BC):
                b = cc % 2
                process(mm, cc, b, drain=(cc >= 2))
                nxt = cc + 2
                if static:
                    if nxt < BC:
                        fire(mm, nxt, b)
                    else:
                        fire(1 - mm, nxt - BC, b)
                else:
                    @pl.when(m * BC + nxt < nchunk)
                    def _():
                        if nxt < BC:
                            fire(mm, nxt, b)
                        else:
                            fire(1 - mm, nxt - BC, b)

        pltpu.sync_copy(idx_hbm.at[pl.ds(ibase, 2 * BC)], idx0)
        fire(0, 0, 0)
        fire(0, 1, 1)
        peel = 2 if nblk % 2 == 0 else 1
        block(0, 0, True, first=True)
        if peel == 2:
            block(1, 1, True)

        @pl.loop(0, (nblk - peel) // 2)
        def _(i):
            for mmi in range(2):
                block(2 * i + peel + mmi, (peel + mmi) % 2, False)

        drain_scatters(0)
        drain_scatters(1)
        plsc.subcore_barrier()
        pltpu.sync_copy(accn.at[pl.ds(row0, rpt)],
                        outn_hbm.at[c, pl.ds(row0, rpt)])
        pltpu.sync_copy(accd.at[pl.ds(row0, rpt)],
                        outd_hbm.at[c, pl.ds(row0, rpt)])

    return edge_kernel


def kernel(x, edge_index, W1, b1, beta, W2, b2):
    n, d = x.shape
    e = edge_index.shape[1]
    n2 = ((n + NS * 8 - 1) // (NS * 8)) * (NS * 8)
    step = NW * CH * 2 * BC
    e_pad = ((e + step - 1) // step) * step
    ept = e_pad // NW
    nchunk = ept // CH

    nb = 25
    bn = n // nb
    t40, t32 = pl.pallas_call(
        _encode_body,
        grid=(nb,),
        in_specs=[
            pl.BlockSpec((bn, d), lambda i: (i, 0)),
            pl.BlockSpec((d, H), lambda i: (0, 0)),
            pl.BlockSpec((1, H), lambda i: (0, 0)),
        ],
        out_specs=[
            pl.BlockSpec((bn, RS), lambda i: (i, 0)),
            pl.BlockSpec((bn, H), lambda i: (i, 0)),
        ],
        out_shape=[
            jax.ShapeDtypeStruct((n, RS), jnp.float32),
            jax.ShapeDtypeStruct((n, H), jnp.float32),
        ],
    )(x, W1.T, b1.reshape(1, H))
    t40_full = jnp.pad(t40, ((0, n2 - n), (0, 0)))
    t32_full = jnp.pad(t32, ((0, n2 - n), (0, 0)))

    src = jnp.concatenate(
        [edge_index[0], jnp.zeros((e_pad - e,), jnp.int32)])
    dst = jnp.concatenate(
        [edge_index[1], jnp.full((e_pad - e,), n2 - 1, jnp.int32)])
    idx = jnp.stack([src.reshape(NW, nchunk, CH),
                     dst.reshape(NW, nchunk, CH)], axis=2)
    idx = idx.reshape(NW * nchunk * 2, CH)

    bf = beta.astype(jnp.float32)
    par = jnp.concatenate(
        [jnp.full((L,), bf), jnp.full((L,), jnp.abs(bf))])

    pn, pd = _make_edge_kernel(n2, nchunk)(t40_full, t32_full, idx, par)

    par2 = jnp.stack([bf, jnp.abs(bf)]).reshape(1, 2)
    out = pl.pallas_call(
        _finish_body,
        grid=(nb,),
        in_specs=[
            pl.BlockSpec((bn, RS), lambda i: (i, 0)),
            pl.BlockSpec((bn, H), lambda i: (i, 0)),
            pl.BlockSpec((bn, H), lambda i: (i, 0)),
            pl.BlockSpec((bn, 1), lambda i: (i, 0)),
            pl.BlockSpec((bn, 1), lambda i: (i, 0)),
            pl.BlockSpec((1, 2), lambda i: (0, 0)),
            pl.BlockSpec((H, 2), lambda i: (0, 0)),
            pl.BlockSpec((1, 2), lambda i: (0, 0)),
        ],
        out_specs=pl.BlockSpec((bn, 2), lambda i: (i, 0)),
        out_shape=jax.ShapeDtypeStruct((n, 2), jnp.float32),
    )(t40, pn[0, :n], pn[1, :n], pd[0, :n].reshape(n, 1),
      pd[1, :n].reshape(n, 1), par2, W2.T, b2.reshape(1, 2))
    return out
